# baseline (speedup 1.0000x reference)
import jax
import jax.numpy as jnp
from jax import lax
from jax.experimental import pallas as pl
from jax.experimental.pallas import tpu as pltpu
from jax.experimental.pallas import tpu_sc as plsc

B = 16384
L = 200
V = 512
D = 16
NW = 32
RPW = B // NW
CHUNK = 32
NCH = RPW // CHUNK
GP = CHUNK // 16
UNROLL = 8

_mesh = plsc.VectorSubcoreMesh(core_axis_name="c", subcore_axis_name="s")

_OUT_TYPE = jax.ShapeDtypeStruct((B,), jnp.float32)
_SCRATCH = [
    pltpu.VMEM((CHUNK, L), jnp.int32),
    pltpu.VMEM((CHUNK, L), jnp.int32),
    pltpu.VMEM((CHUNK, L), jnp.int32),
    pltpu.VMEM((CHUNK, L), jnp.int32),
    pltpu.VMEM((D * V,), jnp.float32),
    pltpu.VMEM((D * 16,), jnp.float32),
    pltpu.VMEM((16,), jnp.float32),
    pltpu.VMEM((16 * V,), jnp.float32),
    pltpu.VMEM((RPW,), jnp.float32),
    pltpu.SemaphoreType.DMA,
    pltpu.SemaphoreType.DMA,
]


def _sc_body(ids_hbm, mask_hbm, embt_hbm, wb_hbm, bias_hbm, out_hbm,
             ids_a, ids_b, mask_a, mask_b, embt_v, wb_v, bias_v, s_v, out_v,
             sem_a, sem_b):
    wid = lax.axis_index("s") * 2 + lax.axis_index("c")
    base = wid * RPW
    lanes = lax.iota(jnp.int32, 16)

    pltpu.sync_copy(embt_hbm, embt_v)
    pltpu.sync_copy(wb_hbm, wb_v)
    pltpu.sync_copy(bias_hbm, bias_v)

    wvecs = [wb_v[pl.ds(d * 16, 16)] for d in range(D)]
    for g in range(V // 16):
        acc = jnp.zeros((16,), jnp.float32)
        for d in range(D):
            acc = acc + embt_v[pl.ds(d * V + g * 16, 16)] * wvecs[d]
        for j in range(16):
            s_v[pl.ds(j * V + g * 16, 16)] = acc

    bias_vec = bias_v[...]
    lane_off = lanes * V

    def start(ch, idbuf, mkbuf, sem):
        r0 = base + ch * CHUNK
        h1 = pltpu.async_copy(ids_hbm.at[pl.ds(r0, CHUNK), :], idbuf, sem)
        h2 = pltpu.async_copy(mask_hbm.at[pl.ds(r0, CHUNK), :], mkbuf, sem)
        return h1, h2

    bufs = [(ids_a, mask_a, sem_a), (ids_b, mask_b, sem_b)]
    pending = start(0, *bufs[0])
    for ch in range(NCH):
        nxt = start(ch + 1, *bufs[(ch + 1) % 2]) if ch + 1 < NCH else None
        pending[0].wait()
        pending[1].wait()
        idbuf, mkbuf, _ = bufs[ch % 2]
        for g in range(GP):
            row_vec = lanes + g * 16

            def body(it, carry):
                acc, cnt, pos = carry
                for k in range(UNROLL):
                    p = pos + k
                    idv = plsc.load_gather(idbuf, [row_vec, p])
                    mv = plsc.load_gather(mkbuf, [row_vec, p])
                    sval = plsc.load_gather(s_v, [idv + lane_off])
                    acc = acc + sval * mv.astype(jnp.float32)
                    cnt = cnt + mv
                return acc, cnt, pos + UNROLL

            acc, cnt, _ = lax.fori_loop(
                0, L // UNROLL, body,
                (jnp.zeros((16,), jnp.float32), jnp.zeros((16,), jnp.int32),
                 jnp.zeros((16,), jnp.int32)))
            denom = jnp.maximum(cnt.astype(jnp.float32), 1.0)
            inv = 1.0 / denom
            inv = inv * (2.0 - denom * inv)
            out_v[pl.ds((ch * GP + g) * 16, 16)] = acc * inv + bias_vec
        pending = nxt

    pltpu.sync_copy(out_v, out_hbm.at[pl.ds(base, RPW)])


_sc_pool = pl.kernel(
    _sc_body,
    out_type=_OUT_TYPE,
    mesh=_mesh,
    compiler_params=pltpu.CompilerParams(needs_layout_passes=False),
    scratch_types=_SCRATCH,
)


def kernel(input_ids, attention_mask, emb_weight, proj_weight, proj_bias):
    ids = input_ids.astype(jnp.int32)
    mask = attention_mask.astype(jnp.int32)
    embt = emb_weight.astype(jnp.float32).T.reshape(D * V)
    wb = jnp.broadcast_to(
        proj_weight.astype(jnp.float32).reshape(D, 1), (D, 16)).reshape(D * 16)
    bias = jnp.broadcast_to(proj_bias.astype(jnp.float32), (16,))
    out = _sc_pool(ids, mask, embt, wb, bias)
    return out.reshape(B, 1)


# speedup vs baseline: 1.2897x; 1.2897x over previous
import jax
import jax.numpy as jnp
from jax import lax
from jax.experimental import pallas as pl
from jax.experimental.pallas import tpu as pltpu
from jax.experimental.pallas import tpu_sc as plsc

B = 16384
L = 200
V = 512
D = 16
NW = 32
RPW = B // NW
CHUNK = 32
NCH = RPW // CHUNK
GP = CHUNK // 16
UNROLL = 8

_mesh = plsc.VectorSubcoreMesh(core_axis_name="c", subcore_axis_name="s")

_OUT_TYPE = jax.ShapeDtypeStruct((B,), jnp.float32)
_SCRATCH = [
    pltpu.VMEM((CHUNK * L,), jnp.int32),
    pltpu.VMEM((CHUNK * L,), jnp.int32),
    pltpu.VMEM((CHUNK * L,), jnp.int32),
    pltpu.VMEM((CHUNK * L,), jnp.int32),
    pltpu.VMEM((D * V,), jnp.float32),
    pltpu.VMEM((D * 16,), jnp.float32),
    pltpu.VMEM((16,), jnp.float32),
    pltpu.VMEM((16 * V,), jnp.float32),
    pltpu.VMEM((RPW,), jnp.float32),
    pltpu.SemaphoreType.DMA,
    pltpu.SemaphoreType.DMA,
]


def _sc_body(ids_hbm, mask_hbm, embt_hbm, wb_hbm, bias_hbm, out_hbm,
             ids_a, ids_b, mask_a, mask_b, embt_v, wb_v, bias_v, s_v, out_v,
             sem_a, sem_b):
    wid = lax.axis_index("s") * 2 + lax.axis_index("c")
    base = wid * RPW
    lanes = lax.iota(jnp.int32, 16)

    pltpu.sync_copy(embt_hbm, embt_v)
    pltpu.sync_copy(wb_hbm, wb_v)
    pltpu.sync_copy(bias_hbm, bias_v)

    wvecs = [wb_v[pl.ds(d * 16, 16)] for d in range(D)]
    for g in range(V // 16):
        acc = jnp.zeros((16,), jnp.float32)
        for d in range(D):
            acc = acc + embt_v[pl.ds(d * V + g * 16, 16)] * wvecs[d]
        for j in range(16):
            s_v[pl.ds(j * V + g * 16, 16)] = acc

    bias_vec = bias_v[...]
    lane_off = lanes * V

    def start(ch, idbuf, mkbuf, sem):
        e0 = (base + ch * CHUNK) * L
        h1 = pltpu.async_copy(ids_hbm.at[pl.ds(e0, CHUNK * L)], idbuf, sem)
        h2 = pltpu.async_copy(mask_hbm.at[pl.ds(e0, CHUNK * L)], mkbuf, sem)
        return h1, h2

    bufs = [(ids_a, mask_a, sem_a), (ids_b, mask_b, sem_b)]
    pending = start(0, *bufs[0])
    for ch in range(NCH):
        nxt = start(ch + 1, *bufs[(ch + 1) % 2]) if ch + 1 < NCH else None
        pending[0].wait()
        pending[1].wait()
        idbuf, mkbuf, _ = bufs[ch % 2]
        for g in range(GP):
            pos0 = (lanes + g * 16) * L

            def body(it, carry):
                acc, cnt, pos = carry
                for k in range(UNROLL):
                    p = pos + k
                    idv = plsc.load_gather(idbuf, [p])
                    mv = plsc.load_gather(mkbuf, [p])
                    sval = plsc.load_gather(s_v, [idv + lane_off])
                    acc = acc + sval * mv.astype(jnp.float32)
                    cnt = cnt + mv
                return acc, cnt, pos + UNROLL

            acc, cnt, _ = lax.fori_loop(
                0, L // UNROLL, body,
                (jnp.zeros((16,), jnp.float32), jnp.zeros((16,), jnp.int32),
                 pos0))
            denom = jnp.maximum(cnt.astype(jnp.float32), 1.0)
            inv = 1.0 / denom
            inv = inv * (2.0 - denom * inv)
            out_v[pl.ds((ch * GP + g) * 16, 16)] = acc * inv + bias_vec
        pending = nxt

    pltpu.sync_copy(out_v, out_hbm.at[pl.ds(base, RPW)])


_sc_pool = pl.kernel(
    _sc_body,
    out_type=_OUT_TYPE,
    mesh=_mesh,
    compiler_params=pltpu.CompilerParams(needs_layout_passes=False),
    scratch_types=_SCRATCH,
)


def kernel(input_ids, attention_mask, emb_weight, proj_weight, proj_bias):
    ids = input_ids.astype(jnp.int32).reshape(B * L)
    mask = attention_mask.astype(jnp.int32).reshape(B * L)
    embt = emb_weight.astype(jnp.float32).T.reshape(D * V)
    wb = jnp.broadcast_to(
        proj_weight.astype(jnp.float32).reshape(D, 1), (D, 16)).reshape(D * 16)
    bias = jnp.broadcast_to(proj_bias.astype(jnp.float32), (16,))
    out = _sc_pool(ids, mask, embt, wb, bias)
    return out.reshape(B, 1)


# speedup vs baseline: 1.3206x; 1.0240x over previous
import jax
import jax.numpy as jnp
from jax import lax
from jax.experimental import pallas as pl
from jax.experimental.pallas import tpu as pltpu
from jax.experimental.pallas import tpu_sc as plsc

B = 16384
L = 200
V = 512
D = 16
NW = 32
RPW = B // NW
CHUNK = 32
NCH = RPW // CHUNK
GP = CHUNK // 16
UNROLL = 8
REG = 1024

_mesh = plsc.VectorSubcoreMesh(core_axis_name="c", subcore_axis_name="s")

_OUT_TYPE = jax.ShapeDtypeStruct((B,), jnp.float32)
_SCRATCH = [
    pltpu.VMEM((CHUNK * L,), jnp.int32),
    pltpu.VMEM((CHUNK * L,), jnp.int32),
    pltpu.VMEM((D * V,), jnp.float32),
    pltpu.VMEM((D * 16,), jnp.float32),
    pltpu.VMEM((16,), jnp.float32),
    pltpu.VMEM((16 * REG,), jnp.float32),
    pltpu.VMEM((RPW,), jnp.float32),
    pltpu.SemaphoreType.DMA,
    pltpu.SemaphoreType.DMA,
]


def _sc_body(mid_hbm, embt_hbm, wb_hbm, bias_hbm, out_hbm,
             ids_a, ids_b, embt_v, wb_v, bias_v, s_v, out_v,
             sem_a, sem_b):
    wid = lax.axis_index("s") * 2 + lax.axis_index("c")
    base = wid * RPW
    lanes = lax.iota(jnp.int32, 16)

    pltpu.sync_copy(embt_hbm, embt_v)
    pltpu.sync_copy(wb_hbm, wb_v)
    pltpu.sync_copy(bias_hbm, bias_v)

    wvecs = [wb_v[pl.ds(d * 16, 16)] for d in range(D)]
    for g in range(V // 16):
        acc = jnp.zeros((16,), jnp.float32)
        for d in range(D):
            acc = acc + embt_v[pl.ds(d * V + g * 16, 16)] * wvecs[d]
        for j in range(16):
            s_v[pl.ds(j * REG + g * 16, 16)] = acc
    plsc.store_scatter(s_v, [lanes * REG + V], jnp.zeros((16,), jnp.float32))

    bias_vec = bias_v[...]
    lane_off = lanes * REG

    def start(ch, buf, sem):
        e0 = (base + ch * CHUNK) * L
        return pltpu.async_copy(mid_hbm.at[pl.ds(e0, CHUNK * L)], buf, sem)

    bufs = [(ids_a, sem_a), (ids_b, sem_b)]
    pending = start(0, *bufs[0])
    for ch in range(NCH):
        nxt = start(ch + 1, *bufs[(ch + 1) % 2]) if ch + 1 < NCH else None
        pending.wait()
        buf, _ = bufs[ch % 2]
        for g in range(GP):
            pos0 = (lanes + g * 16) * L

            def body(it, carry):
                acc, cnt, pos = carry
                for k in range(UNROLL):
                    mid = plsc.load_gather(buf, [pos + k])
                    sval = plsc.load_gather(s_v, [mid + lane_off])
                    acc = acc + sval
                    cnt = cnt + (mid < V).astype(jnp.int32)
                return acc, cnt, pos + UNROLL

            acc, cnt, _ = lax.fori_loop(
                0, L // UNROLL, body,
                (jnp.zeros((16,), jnp.float32), jnp.zeros((16,), jnp.int32),
                 pos0))
            denom = jnp.maximum(cnt.astype(jnp.float32), 1.0)
            inv = 1.0 / denom
            inv = inv * (2.0 - denom * inv)
            out_v[pl.ds((ch * GP + g) * 16, 16)] = acc * inv + bias_vec
        pending = nxt

    pltpu.sync_copy(out_v, out_hbm.at[pl.ds(base, RPW)])


_sc_pool = pl.kernel(
    _sc_body,
    out_type=_OUT_TYPE,
    mesh=_mesh,
    compiler_params=pltpu.CompilerParams(needs_layout_passes=False),
    scratch_types=_SCRATCH,
)


def kernel(input_ids, attention_mask, emb_weight, proj_weight, proj_bias):
    mid = jnp.where(attention_mask != 0, input_ids.astype(jnp.int32), V)
    mid = mid.reshape(B * L)
    embt = emb_weight.astype(jnp.float32).T.reshape(D * V)
    wb = jnp.broadcast_to(
        proj_weight.astype(jnp.float32).reshape(D, 1), (D, 16)).reshape(D * 16)
    bias = jnp.broadcast_to(proj_bias.astype(jnp.float32), (16,))
    out = _sc_pool(mid, embt, wb, bias)
    return out.reshape(B, 1)


# speedup vs baseline: 1.3302x; 1.0072x over previous
import jax
import jax.numpy as jnp
from jax import lax
from jax.experimental import pallas as pl
from jax.experimental.pallas import tpu as pltpu
from jax.experimental.pallas import tpu_sc as plsc

B = 16384
L = 200
V = 512
D = 16
NW = 32
RPW = B // NW
CHUNK = 32
NCH = RPW // CHUNK
GP = CHUNK // 16
UNROLL = 8
REG = 1024

_mesh = plsc.VectorSubcoreMesh(core_axis_name="c", subcore_axis_name="s")

_OUT_TYPE = jax.ShapeDtypeStruct((B,), jnp.float32)
_SCRATCH = [
    pltpu.VMEM((CHUNK * L,), jnp.int32),
    pltpu.VMEM((CHUNK * L,), jnp.int32),
    pltpu.VMEM((D * V,), jnp.float32),
    pltpu.VMEM((D * 16,), jnp.float32),
    pltpu.VMEM((16,), jnp.float32),
    pltpu.VMEM((16 * REG,), jnp.float32),
    pltpu.VMEM((RPW,), jnp.float32),
    pltpu.SemaphoreType.DMA,
    pltpu.SemaphoreType.DMA,
]


def _sc_body(mid_hbm, embt_hbm, wb_hbm, bias_hbm, out_hbm,
             ids_a, ids_b, embt_v, wb_v, bias_v, s_v, out_v,
             sem_a, sem_b):
    wid = lax.axis_index("s") * 2 + lax.axis_index("c")
    base = wid * RPW
    lanes = lax.iota(jnp.int32, 16)

    pltpu.sync_copy(embt_hbm, embt_v)
    pltpu.sync_copy(wb_hbm, wb_v)
    pltpu.sync_copy(bias_hbm, bias_v)

    wvecs = [wb_v[pl.ds(d * 16, 16)] for d in range(D)]
    for g in range(V // 16):
        acc = jnp.zeros((16,), jnp.float32)
        for d in range(D):
            acc = acc + embt_v[pl.ds(d * V + g * 16, 16)] * wvecs[d]
        for j in range(16):
            s_v[pl.ds(j * REG + g * 16, 16)] = acc
    plsc.store_scatter(s_v, [lanes * REG + V], jnp.zeros((16,), jnp.float32))

    bias_vec = bias_v[...]
    lane_off = lanes * REG

    def start(ch, buf, sem):
        e0 = (base + ch * CHUNK) * L
        return pltpu.async_copy(mid_hbm.at[pl.ds(e0, CHUNK * L)], buf, sem)

    bufs = [(ids_a, sem_a), (ids_b, sem_b)]
    pending = start(0, *bufs[0])
    for ch in range(NCH):
        nxt = start(ch + 1, *bufs[(ch + 1) % 2]) if ch + 1 < NCH else None
        pending.wait()
        buf, _ = bufs[ch % 2]
        for g in range(GP):
            pos0 = (lanes + g * 16) * L

            zf = jnp.zeros((16,), jnp.float32)
            zi = jnp.zeros((16,), jnp.int32)

            @plsc.parallel_loop(0, L, 1, unroll=UNROLL,
                                carry=((zf, zf, zf, zf), (zi, zi, zi, zi)))
            def loop(i, carry):
                accs, invs = carry
                mid = plsc.load_gather(buf, [pos0 + i])
                sval = plsc.load_gather(s_v, [mid + lane_off])
                return ((accs[1], accs[2], accs[3], accs[0] + sval),
                        (invs[1], invs[2], invs[3], invs[0] + (mid >> 9)))

            accs, invs = loop
            acc = (accs[0] + accs[1]) + (accs[2] + accs[3])
            ninv = (invs[0] + invs[1]) + (invs[2] + invs[3])
            cnt = L - ninv
            denom = jnp.maximum(cnt.astype(jnp.float32), 1.0)
            inv = 1.0 / denom
            inv = inv * (2.0 - denom * inv)
            out_v[pl.ds((ch * GP + g) * 16, 16)] = acc * inv + bias_vec
        pending = nxt

    pltpu.sync_copy(out_v, out_hbm.at[pl.ds(base, RPW)])


_sc_pool = pl.kernel(
    _sc_body,
    out_type=_OUT_TYPE,
    mesh=_mesh,
    compiler_params=pltpu.CompilerParams(needs_layout_passes=False),
    scratch_types=_SCRATCH,
)


def kernel(input_ids, attention_mask, emb_weight, proj_weight, proj_bias):
    mid = jnp.where(attention_mask != 0, input_ids.astype(jnp.int32), V)
    mid = mid.reshape(B * L)
    embt = emb_weight.astype(jnp.float32).T.reshape(D * V)
    wb = jnp.broadcast_to(
        proj_weight.astype(jnp.float32).reshape(D, 1), (D, 16)).reshape(D * 16)
    bias = jnp.broadcast_to(proj_bias.astype(jnp.float32), (16,))
    out = _sc_pool(mid, embt, wb, bias)
    return out.reshape(B, 1)
